# baseline (speedup 1.0000x reference)
import jax
import jax.numpy as jnp
from jax import lax
from jax.experimental import pallas as pl
from jax.experimental.pallas import tpu as pltpu
from jax.experimental.pallas import tpu_sc as plsc

EMB = 100000
FEAT = 128
NTOT = 819200

_info = plsc.get_sparse_core_info()
NC, NS, L = _info.num_cores, _info.num_subcores, _info.num_lanes
NW = NC * NS
PER_W = NTOT // NW
B = 128
CHUNKS = PER_W // B
DEPTH = 3


def _body(pos_hbm, pairs_hbm, out_hbm, pos_all,
          il0, il1, il2, lw0, lw1, lw2,
          pb0, pb1, pb2,
          gs0, gs1, gs2, os0, os1, os2):
    wid = lax.axis_index("s") * NC + lax.axis_index("c")
    base = wid * PER_W

    idx = (il0, il1, il2)
    lwb = (lw0, lw1, lw2)
    pairs = (pb0, pb1, pb2)
    gsem = (gs0, gs1, gs2)
    osem = (os0, os1, os2)

    pltpu.sync_copy(pos_hbm.at[pl.ds(base, PER_W)], pos_all)

    def prep(g, s):
        @pl.when(jnp.logical_and(g >= DEPTH, g < CHUNKS))
        def _():
            pltpu.make_async_copy(
                pairs[s].at[:, pl.ds(0, FEAT)],
                out_hbm.at[pl.ds(base, B)], osem[s]).wait()

        @pl.when(g < CHUNKS)
        def _():
            for j in range(B // L):
                sl = pl.ds(j * L, L)
                data = pos_all[pl.ds(g * B + j * L, L)] * float(EMB)
                li = jnp.clip(data, 0.0, float(EMB - 1)).astype(jnp.int32)
                li = jnp.minimum(li, EMB - 2)
                idx[s][sl] = li
                lwb[s][sl] = jnp.clip(data - li.astype(jnp.float32), 0.0, 1.0)
            pltpu.async_copy(pairs_hbm.at[idx[s]], pairs[s], gsem[s])

    def consume(g, s):
        @pl.when(g < CHUNKS)
        def _():
            pltpu.make_async_copy(
                pairs_hbm.at[idx[s]], pairs[s], gsem[s]).wait()

            def lerp_group(bb, c):
                lw16 = lwb[s][pl.ds(bb * L, L)]
                for k in range(L):
                    b = bb * L + k
                    lwv = jnp.full((L,), lw16[k], jnp.float32)
                    rwv = 1.0 - lwv
                    for j in range(FEAT // L):
                        sl = pl.ds(j * L, L)
                        slr = pl.ds(FEAT + j * L, L)
                        pairs[s][b, sl] = (rwv * pairs[s][b, sl]
                                           + lwv * pairs[s][b, slr])
                return c

            lax.fori_loop(0, B // L, lerp_group, 0)
            pltpu.async_copy(
                pairs[s].at[:, pl.ds(0, FEAT)],
                out_hbm.at[pl.ds(base + g * B, B)], osem[s])

    prep(0, 0)
    prep(1, 1)

    def tri_step(i, carry):
        for k in range(DEPTH):
            g = DEPTH * i + k
            prep(g + 2, (k + 2) % DEPTH)
            consume(g, k)
        return carry

    lax.fori_loop(0, (CHUNKS + DEPTH - 1) // DEPTH, tri_step, 0)

    for s in range(DEPTH):
        pltpu.make_async_copy(
            pairs[s].at[:, pl.ds(0, FEAT)],
            out_hbm.at[pl.ds(base, B)], osem[s]).wait()


def kernel(seq_positions, lookup_weight):
    pair_table = jnp.concatenate(
        [lookup_weight[:-1], lookup_weight[1:]], axis=1)
    mesh = plsc.VectorSubcoreMesh(core_axis_name="c", subcore_axis_name="s")
    k = pl.kernel(
        _body,
        mesh=mesh,
        out_type=jax.ShapeDtypeStruct((NTOT, FEAT), jnp.float32),
        scratch_types=(
            [pltpu.VMEM((PER_W,), jnp.float32)]
            + [pltpu.VMEM((B,), jnp.int32) for _ in range(DEPTH)]
            + [pltpu.VMEM((B,), jnp.float32) for _ in range(DEPTH)]
            + [pltpu.VMEM((B, 2 * FEAT), jnp.float32) for _ in range(DEPTH)]
            + [pltpu.SemaphoreType.DMA for _ in range(2 * DEPTH)]
        ),
    )
    return k(seq_positions, pair_table)


# speedup vs baseline: 1.2767x; 1.2767x over previous
import jax
import jax.numpy as jnp
from jax import lax
from jax.experimental import pallas as pl
from jax.experimental.pallas import tpu as pltpu
from jax.experimental.pallas import tpu_sc as plsc

EMB = 100000
FEAT = 128
NTOT = 819200

_info = plsc.get_sparse_core_info()
NC, NS, L = _info.num_cores, _info.num_subcores, _info.num_lanes
NW = NC * NS
PER_W = NTOT // NW
B = 128
CHUNKS = PER_W // B
DEPTH = 3


def _body(pos_hbm, table_hbm, out_hbm, pos_all,
          il0, il1, il2, ir0, ir1, ir2, lw0, lw1, lw2,
          rl0, rl1, rl2, rr0, rr1, rr2,
          gs0, gs1, gs2, os0, os1, os2):
    wid = lax.axis_index("s") * NC + lax.axis_index("c")
    base = wid * PER_W

    idx_l = (il0, il1, il2)
    idx_r = (ir0, ir1, ir2)
    lwb = (lw0, lw1, lw2)
    rows_l = (rl0, rl1, rl2)
    rows_r = (rr0, rr1, rr2)
    gsem = (gs0, gs1, gs2)
    osem = (os0, os1, os2)

    pltpu.sync_copy(pos_hbm.at[pl.ds(base, PER_W)], pos_all)

    def prep(g, s):
        @pl.when(jnp.logical_and(g >= DEPTH, g < CHUNKS))
        def _():
            pltpu.make_async_copy(
                rows_l[s], out_hbm.at[pl.ds(base, B)], osem[s]).wait()

        @pl.when(g < CHUNKS)
        def _():
            for j in range(B // L):
                sl = pl.ds(j * L, L)
                data = pos_all[pl.ds(g * B + j * L, L)] * float(EMB)
                li = jnp.clip(data, 0.0, float(EMB - 1)).astype(jnp.int32)
                idx_l[s][sl] = li
                idx_r[s][sl] = jnp.minimum(li + 1, EMB - 1)
                lwb[s][sl] = data - li.astype(jnp.float32)
            pltpu.async_copy(table_hbm.at[idx_l[s]], rows_l[s], gsem[s])
            pltpu.async_copy(table_hbm.at[idx_r[s]], rows_r[s], gsem[s])

    def consume(g, s):
        @pl.when(g < CHUNKS)
        def _():
            pltpu.make_async_copy(
                table_hbm.at[idx_l[s]], rows_l[s], gsem[s]).wait()
            pltpu.make_async_copy(
                table_hbm.at[idx_r[s]], rows_r[s], gsem[s]).wait()

            def lerp_group(bb, c):
                lw16 = lwb[s][pl.ds(bb * L, L)]
                for k in range(L):
                    b = bb * L + k
                    lwv = jnp.full((L,), lw16[k], jnp.float32)
                    rwv = 1.0 - lwv
                    for j in range(FEAT // L):
                        sl = pl.ds(j * L, L)
                        rows_l[s][b, sl] = (rwv * rows_l[s][b, sl]
                                            + lwv * rows_r[s][b, sl])
                return c

            lax.fori_loop(0, B // L, lerp_group, 0)
            pltpu.async_copy(
                rows_l[s], out_hbm.at[pl.ds(base + g * B, B)], osem[s])

    prep(0, 0)
    prep(1, 1)

    def tri_step(i, carry):
        for k in range(DEPTH):
            g = DEPTH * i + k
            prep(g + 2, (k + 2) % DEPTH)
            consume(g, k)
        return carry

    lax.fori_loop(0, (CHUNKS + DEPTH - 1) // DEPTH, tri_step, 0)

    for s in range(DEPTH):
        pltpu.make_async_copy(
            rows_l[s], out_hbm.at[pl.ds(base, B)], osem[s]).wait()


def kernel(seq_positions, lookup_weight):
    mesh = plsc.VectorSubcoreMesh(core_axis_name="c", subcore_axis_name="s")
    k = pl.kernel(
        _body,
        mesh=mesh,
        out_type=jax.ShapeDtypeStruct((NTOT, FEAT), jnp.float32),
        scratch_types=(
            [pltpu.VMEM((PER_W,), jnp.float32)]
            + [pltpu.VMEM((B,), jnp.int32) for _ in range(2 * DEPTH)]
            + [pltpu.VMEM((B,), jnp.float32) for _ in range(DEPTH)]
            + [pltpu.VMEM((B, FEAT), jnp.float32) for _ in range(2 * DEPTH)]
            + [pltpu.SemaphoreType.DMA for _ in range(2 * DEPTH)]
        ),
    )
    return k(seq_positions, lookup_weight)
